# baseline (speedup 1.0000x reference)
import functools

import jax
import jax.numpy as jnp
from jax import lax
from jax.experimental import pallas as pl
from jax.experimental.pallas import tpu as pltpu
from jax.experimental.pallas import tpu_sc as plsc

BATCH = 16384
HIST = 50
DIM = 16
TOTAL = BATCH * HIST

_info = plsc.get_sparse_core_info()
NC, NS = _info.num_cores, _info.num_subcores
NW = NC * NS
PER_W = TOTAL // NW
CHUNK = 3200
NCHUNK = PER_W // CHUNK


def _make_gather(size):
    mesh = plsc.VectorSubcoreMesh(core_axis_name="c", subcore_axis_name="s")

    @functools.partial(
        pl.kernel,
        out_type=jax.ShapeDtypeStruct((TOTAL, DIM), jnp.float32),
        mesh=mesh,
        scratch_types=[
            pltpu.VMEM((CHUNK,), jnp.int32),
            pltpu.VMEM((CHUNK, DIM), jnp.float32),
            pltpu.SemaphoreType.DMA,
        ],
        compiler_params=pltpu.CompilerParams(use_tc_tiling_on_sc=False),
    )
    def gather(idx_hbm, table_hbm, out_hbm, idx_v, rows_v, sem):
        wid = lax.axis_index("s") * NC + lax.axis_index("c")
        base = wid * PER_W
        for j in range(NCHUNK):
            off = base + j * CHUNK
            pltpu.sync_copy(idx_hbm.at[pl.ds(off, CHUNK)], idx_v)
            pltpu.async_copy(table_hbm.at[idx_v], rows_v, sem).wait()
            pltpu.sync_copy(rows_v, out_hbm.at[pl.ds(off, CHUNK)])

    return gather


_gather = _make_gather(TOTAL)


def kernel(inputs, emb):
    idx_flat = inputs.reshape(TOTAL)
    out_flat = _gather(idx_flat, emb)
    return out_flat.reshape(BATCH, HIST, DIM)


# speedup vs baseline: 1.3464x; 1.3464x over previous
import functools

import jax
import jax.numpy as jnp
from jax import lax
from jax.experimental import pallas as pl
from jax.experimental.pallas import tpu as pltpu
from jax.experimental.pallas import tpu_sc as plsc

BATCH = 16384
HIST = 50
DIM = 16
TOTAL = BATCH * HIST

_info = plsc.get_sparse_core_info()
NC, NS = _info.num_cores, _info.num_subcores
NW = NC * NS
BBLK = 128
NBT = BATCH // BBLK // NW
IDXB = BBLK * HIST

P_SHAPE = (HIST, DIM // 8, BATCH // BBLK, 8, BBLK)


def _make_gather():
    mesh = plsc.VectorSubcoreMesh(core_axis_name="c", subcore_axis_name="s")

    @functools.partial(
        pl.kernel,
        out_type=jax.ShapeDtypeStruct(P_SHAPE, jnp.float32),
        mesh=mesh,
        scratch_types=[
            pltpu.VMEM((IDXB,), jnp.int32),
            pltpu.VMEM((HIST, BBLK), jnp.int32),
            pltpu.VMEM((BBLK, DIM), jnp.float32),
            pltpu.VMEM((2, 8, BBLK), jnp.float32),
            pltpu.SemaphoreType.DMA,
        ],
        compiler_params=pltpu.CompilerParams(
            use_tc_tiling_on_sc=False, needs_layout_passes=False
        ),
    )
    def gather(idx_hbm, table_hbm, out_hbm, idx_v, idxT_v, rows0, tile0, sem_g):
        wid = lax.axis_index("s") * NC + lax.axis_index("c")
        lane = lax.iota(jnp.int32, 16)
        ft_ids = lane // 8
        fi_ids = lane % 8

        for r in range(NBT):
            bt = wid * NBT + r
            pltpu.sync_copy(idx_hbm.at[pl.ds(bt * IDXB, IDXB)], idx_v)

            @pl.loop(0, HIST)
            def _(h):
                for g in range(BBLK // 16):
                    v = plsc.load_gather(idx_v, [(g * 16 + lane) * HIST + h])
                    idxT_v[h, pl.ds(g * 16, 16)] = v

            @pl.loop(0, HIST)
            def _(h):
                pltpu.async_copy(table_hbm.at[idxT_v.at[h]], rows0, sem_g).wait()

                @pl.loop(0, BBLK, unroll=8)
                def _(k):
                    row = rows0[k, :]
                    plsc.store_scatter(
                        tile0,
                        [ft_ids, fi_ids, jnp.full((16,), k, jnp.int32)],
                        row,
                    )

                pltpu.sync_copy(tile0.at[0], out_hbm.at[h, 0, bt])
                pltpu.sync_copy(tile0.at[1], out_hbm.at[h, 1, bt])

    return gather


_gather = _make_gather()


def kernel(inputs, emb):
    p = _gather(inputs.reshape(TOTAL), emb)
    return p.transpose(2, 4, 0, 1, 3).reshape(BATCH, HIST, DIM)


# speedup vs baseline: 1.6961x; 1.2597x over previous
import functools

import jax
import jax.numpy as jnp
from jax import lax
from jax.experimental import pallas as pl
from jax.experimental.pallas import tpu as pltpu
from jax.experimental.pallas import tpu_sc as plsc

BATCH = 16384
HIST = 50
DIM = 16
TOTAL = BATCH * HIST

_info = plsc.get_sparse_core_info()
NC, NS = _info.num_cores, _info.num_subcores
NW = NC * NS
BBLK = 128
NBT = BATCH // BBLK // NW
IDXB = BBLK * HIST

P_SHAPE = (HIST, DIM // 8, BATCH // BBLK, 8, BBLK)


def _make_gather():
    mesh = plsc.VectorSubcoreMesh(core_axis_name="c", subcore_axis_name="s")

    @functools.partial(
        pl.kernel,
        out_type=jax.ShapeDtypeStruct(P_SHAPE, jnp.float32),
        mesh=mesh,
        scratch_types=[
            pltpu.VMEM((IDXB,), jnp.int32),
            pltpu.VMEM((HIST, BBLK), jnp.int32),
            pltpu.VMEM((BBLK, DIM), jnp.float32),
            pltpu.VMEM((BBLK, DIM), jnp.float32),
            pltpu.VMEM((2, 8, BBLK), jnp.float32),
            pltpu.VMEM((2, 8, BBLK), jnp.float32),
            pltpu.SemaphoreType.DMA,
            pltpu.SemaphoreType.DMA,
            pltpu.SemaphoreType.DMA,
        ],
        compiler_params=pltpu.CompilerParams(
            use_tc_tiling_on_sc=False, needs_layout_passes=False
        ),
    )
    def gather(
        idx_hbm,
        table_hbm,
        out_hbm,
        idx_v,
        idxT_v,
        rows0,
        rows1,
        tile0,
        tile1,
        sem_g,
        sem_s0,
        sem_s1,
    ):
        wid = lax.axis_index("s") * NC + lax.axis_index("c")
        lane = lax.iota(jnp.int32, 16)
        ft_ids = lane // 8
        fi_ids = lane % 8
        rows_bufs = (rows0, rows1)
        tile_bufs = (tile0, tile1)
        sems = (sem_s0, sem_s1)

        def fire_gather(h, rbuf):
            pltpu.async_copy(table_hbm.at[idxT_v.at[h]], rbuf, sem_g)

        def wait_gather(h, rbuf):
            pltpu.make_async_copy(table_hbm.at[idxT_v.at[h]], rbuf, sem_g).wait()

        def wait_store(tbuf, sem, bt):
            pltpu.make_async_copy(tbuf, out_hbm.at[0, :, bt], sem).wait()

        for r in range(NBT):
            bt = wid * NBT + r
            pltpu.sync_copy(idx_hbm.at[pl.ds(bt * IDXB, IDXB)], idx_v)

            @pl.loop(0, HIST)
            def _(h):
                for g in range(BBLK // 16):
                    v = plsc.load_gather(idx_v, [(g * 16 + lane) * HIST + h])
                    idxT_v[h, pl.ds(g * 16, 16)] = v

            fire_gather(0, rows_bufs[0])

            @pl.loop(0, HIST, step=2)
            def _(h):
                for p in range(2):
                    hp = h + p
                    R = rows_bufs[p]
                    T = tile_bufs[p]
                    S = sems[p]
                    wait_gather(hp, R)

                    @pl.when(hp + 1 < HIST)
                    def _():
                        fire_gather(hp + 1, rows_bufs[1 - p])

                    @pl.when(hp >= 2)
                    def _():
                        wait_store(T, S, bt)

                    @pl.loop(0, BBLK, unroll=8)
                    def _(k):
                        row = R[k, :]
                        plsc.store_scatter(
                            T,
                            [ft_ids, fi_ids, jnp.full((16,), k, jnp.int32)],
                            row,
                        )

                    pltpu.async_copy(T, out_hbm.at[hp, :, bt], S)

            wait_store(tile_bufs[0], sems[0], bt)
            wait_store(tile_bufs[1], sems[1], bt)

    return gather


_gather = _make_gather()


def kernel(inputs, emb):
    p = _gather(inputs.reshape(TOTAL), emb)
    return p.transpose(2, 4, 0, 1, 3).reshape(BATCH, HIST, DIM)
